# Initial kernel scaffold; baseline (speedup 1.0000x reference)
import jax
import jax.numpy as jnp
from jax.experimental import pallas as pl


def kernel(input, indices):
    raise NotImplementedError("write your pallas kernel here")


# speedup vs baseline: 3.6281x; 3.6281x over previous
import functools

import jax
import jax.numpy as jnp
from jax import lax
from jax.experimental import pallas as pl
from jax.experimental.pallas import tpu as pltpu
from jax.experimental.pallas import tpu_sc as plsc

N, C, Hin, Win = 4, 96, 256, 256
Hout, Wout = 512, 512
R = N * C
HW_IN = Hin * Win
HW_OUT = Hout * Wout
TOTAL = R * HW_IN
OUT_TOTAL = R * HW_OUT
QBINS = 65536
TASKS = OUT_TOTAL // QBINS
NC_SC = 2
NS_SC = 16
NW = NC_SC * NS_SC
TPW = TASKS // NW
CH = 8192
NB = 1552

_mesh = plsc.VectorSubcoreMesh(
    core_axis_name="c", subcore_axis_name="s",
    num_cores=NC_SC, num_subcores=NS_SC)


def _extract(vec16, lane):
    lanes = lax.iota(jnp.int32, 16)
    return lax.reduce_max(
        jnp.where(lanes == lane, vec16, jnp.int32(-2147483648)), axes=(0,))


@functools.partial(
    pl.kernel,
    out_type=jax.ShapeDtypeStruct((OUT_TOTAL,), jnp.float32),
    mesh=_mesh,
    compiler_params=pltpu.CompilerParams(needs_layout_passes=False),
    scratch_types=[
        pltpu.VMEM((QBINS,), jnp.float32),
        pltpu.VMEM((CH,), jnp.int32),
        pltpu.VMEM((CH,), jnp.float32),
        pltpu.VMEM((NB,), jnp.int32),
    ],
)
def _scatter_sorted(keys_hbm, vals_hbm, bounds_hbm, out_hbm,
                    outb, keyb, valb, bndb):
    wid = lax.axis_index("s") * NC_SC + lax.axis_index("c")
    pltpu.sync_copy(bounds_hbm, bndb)

    def task_body(t, _):
        k = wid * TPW + t
        gbase = k * QBINS

        a0 = (k // 16) * 16
        start = _extract(bndb[pl.ds(a0, 16)], k - a0)
        a1 = ((k + 1) // 16) * 16
        end = _extract(bndb[pl.ds(a1, 16)], k + 1 - a1)

        zero = jnp.zeros((16,), jnp.float32)

        def zbody(j, _):
            outb[pl.ds(j * 16, 16)] = zero
            return 0

        lax.fori_loop(0, QBINS // 16, zbody, 0)

        astart = (start // 8) * 8
        nblk = (end - astart + CH - 1) // CH

        def blk_body(b, _):
            off = jnp.minimum(astart + b * CH, TOTAL - CH)
            pltpu.sync_copy(keys_hbm.at[pl.ds(off, CH)], keyb)
            pltpu.sync_copy(vals_hbm.at[pl.ds(off, CH)], valb)

            def g(j, _):
                kv = keyb[pl.ds(j * 16, 16)]
                vv = valb[pl.ds(j * 16, 16)]
                m = (kv >= gbase) & (kv < gbase + QBINS)
                plsc.store_scatter(outb, [kv - gbase], vv, mask=m)
                return 0

            lax.fori_loop(0, CH // 16, g, 0)
            return 0

        lax.fori_loop(0, nblk, blk_body, 0)
        pltpu.sync_copy(outb, out_hbm.at[pl.ds(gbase, QBINS)])
        return 0

    lax.fori_loop(0, TPW, task_body, 0)


def kernel(input, indices):
    idx = indices.reshape(R, HW_IN).astype(jnp.int32)
    keys = (jnp.arange(R, dtype=jnp.int32)[:, None] * HW_OUT + idx).reshape(-1)
    flat_vals = input.reshape(TOTAL)
    sk, sv = lax.sort((keys, flat_vals), dimension=0, num_keys=1,
                      is_stable=False)
    edges = jnp.arange(TASKS + 1, dtype=jnp.int32) * QBINS
    bounds = jnp.searchsorted(sk, edges, side="left").astype(jnp.int32)
    bounds = jnp.pad(bounds, (0, NB - (TASKS + 1)),
                     constant_values=jnp.int32(TOTAL))
    out = _scatter_sorted(sk, sv, bounds)
    return out.reshape(N, C, Hout, Wout)


# speedup vs baseline: 3.6562x; 1.0077x over previous
import functools

import jax
import jax.numpy as jnp
from jax import lax
from jax.experimental import pallas as pl
from jax.experimental.pallas import tpu as pltpu
from jax.experimental.pallas import tpu_sc as plsc

N, C, Hin, Win = 4, 96, 256, 256
Hout, Wout = 512, 512
R = N * C
HW_IN = Hin * Win
HW_OUT = Hout * Wout
TOTAL = R * HW_IN
OUT_TOTAL = R * HW_OUT
QBINS = 32768
TASKS = OUT_TOTAL // QBINS
NC_SC = 2
NS_SC = 16
NW = NC_SC * NS_SC
TPW = TASKS // NW
CH = 4096
GUNROLL = 8
ZUNROLL = 8
NB = 3088

_mesh = plsc.VectorSubcoreMesh(
    core_axis_name="c", subcore_axis_name="s",
    num_cores=NC_SC, num_subcores=NS_SC)


def _extract(vec16, lane):
    lanes = lax.iota(jnp.int32, 16)
    return lax.reduce_max(
        jnp.where(lanes == lane, vec16, jnp.int32(-2147483648)), axes=(0,))


@functools.partial(
    pl.kernel,
    out_type=jax.ShapeDtypeStruct((OUT_TOTAL,), jnp.float32),
    mesh=_mesh,
    compiler_params=pltpu.CompilerParams(needs_layout_passes=False),
    scratch_types=[
        pltpu.VMEM((QBINS,), jnp.float32),
        pltpu.VMEM((QBINS,), jnp.float32),
        pltpu.VMEM((CH,), jnp.int32),
        pltpu.VMEM((CH,), jnp.int32),
        pltpu.VMEM((CH,), jnp.float32),
        pltpu.VMEM((CH,), jnp.float32),
        pltpu.VMEM((NB,), jnp.int32),
        pltpu.SemaphoreType.DMA,
        pltpu.SemaphoreType.DMA,
        pltpu.SemaphoreType.DMA,
        pltpu.SemaphoreType.DMA,
    ],
)
def _scatter_sorted(keys_hbm, vals_hbm, bounds_hbm, out_hbm,
                    outb0, outb1, keyb0, keyb1, valb0, valb1, bndb,
                    si0, si1, so0, so1):
    wid = lax.axis_index("s") * NC_SC + lax.axis_index("c")
    pltpu.sync_copy(bounds_hbm, bndb)

    def do_task(k, outb, so, not_first):
        gbase = k * QBINS

        a0 = (k // 16) * 16
        start = _extract(bndb[pl.ds(a0, 16)], k - a0)
        a1 = ((k + 1) // 16) * 16
        end = _extract(bndb[pl.ds(a1, 16)], k + 1 - a1)

        @pl.when(not_first)
        def _():
            pltpu.make_async_copy(outb, out_hbm.at[pl.ds(0, QBINS)], so).wait()

        zero = jnp.zeros((16,), jnp.float32)

        def zbody(j, _):
            for u in range(ZUNROLL):
                outb[pl.ds(j * (16 * ZUNROLL) + u * 16, 16)] = zero
            return 0

        lax.fori_loop(0, QBINS // (16 * ZUNROLL), zbody, 0)

        astart = (start // 8) * 8
        nblk = (end - astart + CH - 1) // CH
        npair = (nblk + 1) // 2

        def off_of(b):
            return jnp.minimum(astart + b * CH, TOTAL - CH)

        def scat(keyb, valb):
            def g(j, _):
                for u in range(GUNROLL):
                    kv = keyb[pl.ds(j * (16 * GUNROLL) + u * 16, 16)]
                    vv = valb[pl.ds(j * (16 * GUNROLL) + u * 16, 16)]
                    m = (kv >= gbase) & (kv < gbase + QBINS)
                    plsc.store_scatter(outb, [kv - gbase], vv, mask=m)
                return 0
            lax.fori_loop(0, CH // (16 * GUNROLL), g, 0)

        @pl.when(nblk > 0)
        def _():
            pltpu.async_copy(keys_hbm.at[pl.ds(off_of(0), CH)], keyb0, si0)
            pltpu.async_copy(vals_hbm.at[pl.ds(off_of(0), CH)], valb0, si0)

        def pair_body(bb, _):
            b1 = 2 * bb + 1

            @pl.when(b1 < nblk)
            def _():
                pltpu.async_copy(keys_hbm.at[pl.ds(off_of(b1), CH)], keyb1, si1)
                pltpu.async_copy(vals_hbm.at[pl.ds(off_of(b1), CH)], valb1, si1)

            pltpu.make_async_copy(keys_hbm.at[pl.ds(0, CH)], keyb0, si0).wait()
            pltpu.make_async_copy(vals_hbm.at[pl.ds(0, CH)], valb0, si0).wait()
            scat(keyb0, valb0)

            @pl.when(2 * bb + 2 < nblk)
            def _():
                nb0 = 2 * bb + 2
                pltpu.async_copy(keys_hbm.at[pl.ds(off_of(nb0), CH)], keyb0, si0)
                pltpu.async_copy(vals_hbm.at[pl.ds(off_of(nb0), CH)], valb0, si0)

            @pl.when(b1 < nblk)
            def _():
                pltpu.make_async_copy(keys_hbm.at[pl.ds(0, CH)], keyb1, si1).wait()
                pltpu.make_async_copy(vals_hbm.at[pl.ds(0, CH)], valb1, si1).wait()
                scat(keyb1, valb1)

            return 0

        lax.fori_loop(0, npair, pair_body, 0)
        pltpu.async_copy(outb, out_hbm.at[pl.ds(gbase, QBINS)], so)

    def task_pair(tt, _):
        base_k = wid * TPW + 2 * tt
        do_task(base_k, outb0, so0, tt > 0)
        do_task(base_k + 1, outb1, so1, tt > 0)
        return 0

    lax.fori_loop(0, TPW // 2, task_pair, 0)
    pltpu.make_async_copy(outb0, out_hbm.at[pl.ds(0, QBINS)], so0).wait()
    pltpu.make_async_copy(outb1, out_hbm.at[pl.ds(0, QBINS)], so1).wait()


def kernel(input, indices):
    idx = indices.reshape(R, HW_IN).astype(jnp.int32)
    keys = (jnp.arange(R, dtype=jnp.int32)[:, None] * HW_OUT + idx).reshape(-1)
    flat_vals = input.reshape(TOTAL)
    sk, sv = lax.sort((keys, flat_vals), dimension=0, num_keys=1,
                      is_stable=False)
    edges = jnp.arange(TASKS + 1, dtype=jnp.int32) * QBINS
    bounds = jnp.searchsorted(sk, edges, side="left").astype(jnp.int32)
    bounds = jnp.pad(bounds, (0, NB - (TASKS + 1)),
                     constant_values=jnp.int32(TOTAL))
    out = _scatter_sorted(sk, sv, bounds)
    return out.reshape(N, C, Hout, Wout)


# speedup vs baseline: 3.8392x; 1.0501x over previous
import functools

import jax
import jax.numpy as jnp
from jax import lax
from jax.experimental import pallas as pl
from jax.experimental.pallas import tpu as pltpu
from jax.experimental.pallas import tpu_sc as plsc

N, C, Hin, Win = 4, 96, 256, 256
Hout, Wout = 512, 512
R = N * C
HW_IN = Hin * Win
HW_OUT = Hout * Wout
TOTAL = R * HW_IN
OUT_TOTAL = R * HW_OUT
QBINS = 32768
TASKS = OUT_TOTAL // QBINS
NC_SC = 2
NS_SC = 16
NW = NC_SC * NS_SC
TPW = TASKS // NW
CH = 4096
GUNROLL = 8
ZUNROLL = 8
NEDGE = 112
NGRP = NEDGE // 16
BSROUNDS = 25

_mesh = plsc.VectorSubcoreMesh(
    core_axis_name="c", subcore_axis_name="s",
    num_cores=NC_SC, num_subcores=NS_SC)


def _extract(vec16, lane):
    lanes = lax.iota(jnp.int32, 16)
    return lax.reduce_max(
        jnp.where(lanes == lane, vec16, jnp.int32(-2147483648)), axes=(0,))


@functools.partial(
    pl.kernel,
    out_type=jax.ShapeDtypeStruct((OUT_TOTAL,), jnp.float32),
    mesh=_mesh,
    compiler_params=pltpu.CompilerParams(needs_layout_passes=False),
    scratch_types=[
        pltpu.VMEM((QBINS,), jnp.float32),
        pltpu.VMEM((QBINS,), jnp.float32),
        pltpu.VMEM((CH,), jnp.int32),
        pltpu.VMEM((CH,), jnp.int32),
        pltpu.VMEM((CH,), jnp.float32),
        pltpu.VMEM((CH,), jnp.float32),
        pltpu.VMEM((NEDGE,), jnp.int32),
        pltpu.VMEM((NEDGE,), jnp.int32),
        pltpu.VMEM((NEDGE,), jnp.int32),
        pltpu.SemaphoreType.DMA,
        pltpu.SemaphoreType.DMA,
        pltpu.SemaphoreType.DMA,
        pltpu.SemaphoreType.DMA,
        pltpu.SemaphoreType.DMA,
    ],
)
def _scatter_sorted(keys_hbm, vals_hbm, out_hbm,
                    outb0, outb1, keyb0, keyb1, valb0, valb1,
                    midb, gatb, bndb, si0, si1, so0, so1, sb):
    wid = lax.axis_index("s") * NC_SC + lax.axis_index("c")
    lanes = lax.iota(jnp.int32, 16)

    evals = [
        jnp.minimum(wid * TPW + 16 * g + lanes, jnp.int32(TASKS)) * QBINS
        for g in range(NGRP)
    ]

    def bs_round(_, carry):
        los, his = carry
        nlos, nhis = [], []
        for g in range(NGRP):
            midb[pl.ds(16 * g, 16)] = (los[g] + his[g]) // 2
        pltpu.async_copy(keys_hbm.at[midb], gatb, sb).wait()
        for g in range(NGRP):
            mid = midb[pl.ds(16 * g, 16)]
            gk = gatb[pl.ds(16 * g, 16)]
            pred = gk < evals[g]
            nlos.append(jnp.where(pred, mid + 1, los[g]))
            nhis.append(jnp.where(pred, his[g], mid))
        return nlos, nhis

    init = ([jnp.zeros((16,), jnp.int32)] * NGRP,
            [jnp.full((16,), TOTAL, jnp.int32)] * NGRP)
    los, _ = lax.fori_loop(0, BSROUNDS, bs_round, init)
    for g in range(NGRP):
        bndb[pl.ds(16 * g, 16)] = los[g]

    def do_task(tl, outb, so, not_first):
        k = wid * TPW + tl
        gbase = k * QBINS

        a0 = (tl // 16) * 16
        start = _extract(bndb[pl.ds(a0, 16)], tl - a0)
        a1 = ((tl + 1) // 16) * 16
        end = _extract(bndb[pl.ds(a1, 16)], tl + 1 - a1)

        @pl.when(not_first)
        def _():
            pltpu.make_async_copy(outb, out_hbm.at[pl.ds(0, QBINS)], so).wait()

        zero = jnp.zeros((16,), jnp.float32)

        def zbody(j, _):
            for u in range(ZUNROLL):
                outb[pl.ds(j * (16 * ZUNROLL) + u * 16, 16)] = zero
            return 0

        lax.fori_loop(0, QBINS // (16 * ZUNROLL), zbody, 0)

        astart = (start // 8) * 8
        nblk = (end - astart + CH - 1) // CH
        npair = (nblk + 1) // 2

        def off_of(b):
            return jnp.minimum(astart + b * CH, TOTAL - CH)

        def scat(keyb, valb):
            def g(j, _):
                for u in range(GUNROLL):
                    kv = keyb[pl.ds(j * (16 * GUNROLL) + u * 16, 16)]
                    vv = valb[pl.ds(j * (16 * GUNROLL) + u * 16, 16)]
                    m = (kv >= gbase) & (kv < gbase + QBINS)
                    plsc.store_scatter(outb, [kv - gbase], vv, mask=m)
                return 0
            lax.fori_loop(0, CH // (16 * GUNROLL), g, 0)

        @pl.when(nblk > 0)
        def _():
            pltpu.async_copy(keys_hbm.at[pl.ds(off_of(0), CH)], keyb0, si0)
            pltpu.async_copy(vals_hbm.at[pl.ds(off_of(0), CH)], valb0, si0)

        def pair_body(bb, _):
            b1 = 2 * bb + 1

            @pl.when(b1 < nblk)
            def _():
                pltpu.async_copy(keys_hbm.at[pl.ds(off_of(b1), CH)], keyb1, si1)
                pltpu.async_copy(vals_hbm.at[pl.ds(off_of(b1), CH)], valb1, si1)

            pltpu.make_async_copy(keys_hbm.at[pl.ds(0, CH)], keyb0, si0).wait()
            pltpu.make_async_copy(vals_hbm.at[pl.ds(0, CH)], valb0, si0).wait()
            scat(keyb0, valb0)

            @pl.when(2 * bb + 2 < nblk)
            def _():
                nb0 = 2 * bb + 2
                pltpu.async_copy(keys_hbm.at[pl.ds(off_of(nb0), CH)], keyb0, si0)
                pltpu.async_copy(vals_hbm.at[pl.ds(off_of(nb0), CH)], valb0, si0)

            @pl.when(b1 < nblk)
            def _():
                pltpu.make_async_copy(keys_hbm.at[pl.ds(0, CH)], keyb1, si1).wait()
                pltpu.make_async_copy(vals_hbm.at[pl.ds(0, CH)], valb1, si1).wait()
                scat(keyb1, valb1)

            return 0

        lax.fori_loop(0, npair, pair_body, 0)
        pltpu.async_copy(outb, out_hbm.at[pl.ds(gbase, QBINS)], so)

    def task_pair(tt, _):
        do_task(2 * tt, outb0, so0, tt > 0)
        do_task(2 * tt + 1, outb1, so1, tt > 0)
        return 0

    lax.fori_loop(0, TPW // 2, task_pair, 0)
    pltpu.make_async_copy(outb0, out_hbm.at[pl.ds(0, QBINS)], so0).wait()
    pltpu.make_async_copy(outb1, out_hbm.at[pl.ds(0, QBINS)], so1).wait()


def kernel(input, indices):
    idx = indices.reshape(R, HW_IN).astype(jnp.int32)
    keys = (jnp.arange(R, dtype=jnp.int32)[:, None] * HW_OUT + idx).reshape(-1)
    flat_vals = input.reshape(TOTAL)
    sk, sv = lax.sort((keys, flat_vals), dimension=0, num_keys=1,
                      is_stable=False)
    out = _scatter_sorted(sk, sv)
    return out.reshape(N, C, Hout, Wout)


# speedup vs baseline: 3.8478x; 1.0022x over previous
import functools

import jax
import jax.numpy as jnp
from jax import lax
from jax.experimental import pallas as pl
from jax.experimental.pallas import tpu as pltpu
from jax.experimental.pallas import tpu_sc as plsc

N, C, Hin, Win = 4, 96, 256, 256
Hout, Wout = 512, 512
R = N * C
HW_IN = Hin * Win
HW_OUT = Hout * Wout
TOTAL = R * HW_IN
OUT_TOTAL = R * HW_OUT
QBINS = 32768
TASKS = OUT_TOTAL // QBINS
NC_SC = 2
NS_SC = 16
NW = NC_SC * NS_SC
TPW = TASKS // NW
CH = 2048
GUNROLL = 8
ZUNROLL = 8
NEDGE = 112
NGRP = NEDGE // 16
BSROUNDS = 25

_mesh = plsc.VectorSubcoreMesh(
    core_axis_name="c", subcore_axis_name="s",
    num_cores=NC_SC, num_subcores=NS_SC)


def _extract(vec16, lane):
    lanes = lax.iota(jnp.int32, 16)
    return lax.reduce_max(
        jnp.where(lanes == lane, vec16, jnp.int32(-2147483648)), axes=(0,))


@functools.partial(
    pl.kernel,
    out_type=jax.ShapeDtypeStruct((OUT_TOTAL,), jnp.float32),
    mesh=_mesh,
    compiler_params=pltpu.CompilerParams(needs_layout_passes=False),
    scratch_types=[
        pltpu.VMEM((QBINS,), jnp.float32),
        pltpu.VMEM((QBINS,), jnp.float32),
        pltpu.VMEM((CH,), jnp.int32),
        pltpu.VMEM((CH,), jnp.int32),
        pltpu.VMEM((CH,), jnp.float32),
        pltpu.VMEM((CH,), jnp.float32),
        pltpu.VMEM((NEDGE,), jnp.int32),
        pltpu.VMEM((NEDGE,), jnp.int32),
        pltpu.VMEM((NEDGE,), jnp.int32),
        pltpu.SemaphoreType.DMA,
        pltpu.SemaphoreType.DMA,
        pltpu.SemaphoreType.DMA,
        pltpu.SemaphoreType.DMA,
        pltpu.SemaphoreType.DMA,
    ],
)
def _scatter_sorted(keys_hbm, vals_hbm, out_hbm,
                    outb0, outb1, keyb0, keyb1, valb0, valb1,
                    midb, gatb, bndb, si0, si1, so0, so1, sb):
    wid = lax.axis_index("s") * NC_SC + lax.axis_index("c")
    lanes = lax.iota(jnp.int32, 16)

    evals = [
        jnp.minimum(wid * TPW + 16 * g + lanes, jnp.int32(TASKS)) * QBINS
        for g in range(NGRP)
    ]

    def bs_round(_, carry):
        los, his = carry
        nlos, nhis = [], []
        for g in range(NGRP):
            midb[pl.ds(16 * g, 16)] = (los[g] + his[g]) // 2
        pltpu.async_copy(keys_hbm.at[midb], gatb, sb).wait()
        for g in range(NGRP):
            mid = midb[pl.ds(16 * g, 16)]
            gk = gatb[pl.ds(16 * g, 16)]
            pred = gk < evals[g]
            nlos.append(jnp.where(pred, mid + 1, los[g]))
            nhis.append(jnp.where(pred, his[g], mid))
        return nlos, nhis

    init = ([jnp.zeros((16,), jnp.int32)] * NGRP,
            [jnp.full((16,), TOTAL, jnp.int32)] * NGRP)
    los, _ = lax.fori_loop(0, BSROUNDS, bs_round, init)
    for g in range(NGRP):
        bndb[pl.ds(16 * g, 16)] = los[g]

    def do_task(tl, outb, so, not_first):
        k = wid * TPW + tl
        gbase = k * QBINS

        a0 = (tl // 16) * 16
        start = _extract(bndb[pl.ds(a0, 16)], tl - a0)
        a1 = ((tl + 1) // 16) * 16
        end = _extract(bndb[pl.ds(a1, 16)], tl + 1 - a1)

        @pl.when(not_first)
        def _():
            pltpu.make_async_copy(outb, out_hbm.at[pl.ds(0, QBINS)], so).wait()

        zero = jnp.zeros((16,), jnp.float32)

        def zbody(j, _):
            for u in range(ZUNROLL):
                outb[pl.ds(j * (16 * ZUNROLL) + u * 16, 16)] = zero
            return 0

        lax.fori_loop(0, QBINS // (16 * ZUNROLL), zbody, 0)

        astart = (start // 8) * 8
        nblk = (end - astart + CH - 1) // CH
        npair = (nblk + 1) // 2

        def off_of(b):
            return jnp.minimum(astart + b * CH, TOTAL - CH)

        def scat(keyb, valb):
            def g(j, _):
                for u in range(GUNROLL):
                    kv = keyb[pl.ds(j * (16 * GUNROLL) + u * 16, 16)]
                    vv = valb[pl.ds(j * (16 * GUNROLL) + u * 16, 16)]
                    rel = kv - gbase
                    m = rel.astype(jnp.uint32) < jnp.uint32(QBINS)
                    plsc.store_scatter(outb, [rel], vv, mask=m)
                return 0
            lax.fori_loop(0, CH // (16 * GUNROLL), g, 0)

        @pl.when(nblk > 0)
        def _():
            pltpu.async_copy(keys_hbm.at[pl.ds(off_of(0), CH)], keyb0, si0)
            pltpu.async_copy(vals_hbm.at[pl.ds(off_of(0), CH)], valb0, si0)

        def pair_body(bb, _):
            b1 = 2 * bb + 1

            @pl.when(b1 < nblk)
            def _():
                pltpu.async_copy(keys_hbm.at[pl.ds(off_of(b1), CH)], keyb1, si1)
                pltpu.async_copy(vals_hbm.at[pl.ds(off_of(b1), CH)], valb1, si1)

            pltpu.make_async_copy(keys_hbm.at[pl.ds(0, CH)], keyb0, si0).wait()
            pltpu.make_async_copy(vals_hbm.at[pl.ds(0, CH)], valb0, si0).wait()
            scat(keyb0, valb0)

            @pl.when(2 * bb + 2 < nblk)
            def _():
                nb0 = 2 * bb + 2
                pltpu.async_copy(keys_hbm.at[pl.ds(off_of(nb0), CH)], keyb0, si0)
                pltpu.async_copy(vals_hbm.at[pl.ds(off_of(nb0), CH)], valb0, si0)

            @pl.when(b1 < nblk)
            def _():
                pltpu.make_async_copy(keys_hbm.at[pl.ds(0, CH)], keyb1, si1).wait()
                pltpu.make_async_copy(vals_hbm.at[pl.ds(0, CH)], valb1, si1).wait()
                scat(keyb1, valb1)

            return 0

        lax.fori_loop(0, npair, pair_body, 0)
        pltpu.async_copy(outb, out_hbm.at[pl.ds(gbase, QBINS)], so)

    def task_pair(tt, _):
        do_task(2 * tt, outb0, so0, tt > 0)
        do_task(2 * tt + 1, outb1, so1, tt > 0)
        return 0

    lax.fori_loop(0, TPW // 2, task_pair, 0)
    pltpu.make_async_copy(outb0, out_hbm.at[pl.ds(0, QBINS)], so0).wait()
    pltpu.make_async_copy(outb1, out_hbm.at[pl.ds(0, QBINS)], so1).wait()


def kernel(input, indices):
    idx = indices.reshape(R, HW_IN).astype(jnp.int32)
    keys = (jnp.arange(R, dtype=jnp.int32)[:, None] * HW_OUT + idx).reshape(-1)
    flat_vals = input.reshape(TOTAL)
    sk, sv = lax.sort((keys, flat_vals), dimension=0, num_keys=1,
                      is_stable=False)
    out = _scatter_sorted(sk, sv)
    return out.reshape(N, C, Hout, Wout)
